# baseline (speedup 1.0000x reference)
import functools

import jax
import jax.numpy as jnp
from jax import lax
from jax.experimental import pallas as pl
from jax.experimental.pallas import tpu as pltpu
from jax.experimental.pallas import tpu_sc as plsc

N = 10000
D = 128
E = 320000
LANES = 16

NC = 2
NS = 16
NW = NC * NS
K = 128
EPAD = 10240
CHUNKS = EPAD // K
SUP = 16
NSUP = CHUNKS // SUP
SUPE = SUP * K
NP = 10240
RPT = NP // NS
KG = K // LANES


def _sc_agg(x, srcf, dst3, eaf):
    mesh = plsc.VectorSubcoreMesh(core_axis_name="c", subcore_axis_name="s")

    @functools.partial(
        pl.kernel,
        out_type=jax.ShapeDtypeStruct((NC, NS, RPT, D), jnp.float32),
        mesh=mesh,
        scratch_types=[
            pltpu.VMEM((SUP, K), jnp.int32),
            pltpu.VMEM((CHUNKS, K), jnp.int32),
            pltpu.VMEM((SUP, K), jnp.float32),
            pltpu.VMEM((K, D), jnp.float32),
            pltpu.VMEM_SHARED((NP, D), jnp.float32),
            pltpu.SemaphoreType.DMA,
        ],
    )
    def k(x_hbm, src_hbm, dst_hbm, ea_hbm, out_hbm,
          src_v, dst_all, ea_v, rows_v, acc_sh, gsem):
        cid = lax.axis_index("c")
        sid = lax.axis_index("s")
        wid = cid * NS + sid

        def zrow(i, carry):
            for r in range(D // LANES):
                rows_v[i, pl.ds(r * LANES, LANES)] = jnp.zeros(
                    (LANES,), jnp.float32)
            return carry
        lax.fori_loop(0, K, zrow, 0)
        for t in range(RPT // K):
            pltpu.sync_copy(rows_v,
                            acc_sh.at[pl.ds(sid * RPT + t * K, K)])

        pltpu.sync_copy(dst_hbm.at[wid], dst_all)
        plsc.subcore_barrier()

        def sup_body(si, carry):
            c0 = si * SUP
            pltpu.sync_copy(src_hbm.at[wid, pl.ds(c0, SUP)], src_v)
            pltpu.sync_copy(ea_hbm.at[wid, pl.ds(c0, SUP)], ea_v)

            def chunk(u, c2):
                c = c0 + u
                pltpu.async_copy(x_hbm.at[src_v.at[u]], rows_v,
                                 gsem).wait()

                def scale(g, c3):
                    eav = ea_v[u, pl.ds(g * LANES, LANES)]
                    for li in range(LANES):
                        a = eav[li]
                        j = g * LANES + li
                        for r in range(D // LANES):
                            sl = pl.ds(r * LANES, LANES)
                            rows_v[j, sl] = rows_v[j, sl] * a
                    return c3
                lax.fori_loop(0, KG, scale, 0)

                pltpu.sync_copy(rows_v, acc_sh.at[dst_all.at[c]],
                                add=True)
                return c2
            lax.fori_loop(0, SUP, chunk, 0)
            return carry
        lax.fori_loop(0, NSUP, sup_body, 0)
        plsc.subcore_barrier()

        pltpu.sync_copy(acc_sh.at[pl.ds(sid * RPT, RPT)],
                        out_hbm.at[cid, sid])

    return k(x, srcf, dst3, eaf).reshape(NC, NP, D)


_BN = 400


def _mm_body_relu(p_ref, w_ref, b_ref, o_ref):
    a = p_ref[0] + p_ref[1]
    h = jnp.dot(a, w_ref[...], preferred_element_type=jnp.float32)
    o_ref[...] = jnp.maximum(h + b_ref[...], 0.0)


def _mm_body_lin(p_ref, w_ref, b_ref, o_ref):
    a = p_ref[0] + p_ref[1]
    h = jnp.dot(a, w_ref[...], preferred_element_type=jnp.float32)
    o_ref[...] = h + b_ref[...]


def _mm(p, w, b, relu):
    body = _mm_body_relu if relu else _mm_body_lin
    return pl.pallas_call(
        body,
        grid=(N // _BN,),
        in_specs=[
            pl.BlockSpec((NC, _BN, D), lambda i: (0, i, 0)),
            pl.BlockSpec((D, D), lambda i: (0, 0)),
            pl.BlockSpec((1, D), lambda i: (0, 0)),
        ],
        out_specs=pl.BlockSpec((_BN, D), lambda i: (i, 0)),
        out_shape=jax.ShapeDtypeStruct((N, D), jnp.float32),
    )(p, w, b.reshape(1, D))


def kernel(x, edge_index, edge_attr, W1, b1, W2, b2):
    pad = NW * EPAD - E
    src = jnp.pad(edge_index[0], (0, pad)).reshape(NW, CHUNKS, K)
    dst = jnp.pad(edge_index[1], (0, pad)).reshape(NW, CHUNKS, K)
    ea = jnp.pad(edge_attr, (0, pad)).reshape(NW, CHUNKS, K)
    p1 = _sc_agg(x, src, dst, ea)
    h = _mm(p1, W1, b1, relu=True)
    p2 = _sc_agg(h, src, dst, ea)
    return _mm(p2, W2, b2, relu=False)


# speedup vs baseline: 2.1161x; 2.1161x over previous
import functools

import jax
import jax.numpy as jnp
from jax import lax
from jax.experimental import pallas as pl
from jax.experimental.pallas import tpu as pltpu
from jax.experimental.pallas import tpu_sc as plsc

N = 10000
D = 128
E = 320000
LANES = 16

NC = 2
NS = 16
NW = NC * NS
EPT = E // NW
K = 80
CHUNKS = EPT // K
NP = 10240
RPT = NP // NS
ZR = 128
KG = K // LANES


def _sc_agg(x, src, dst, ea):
    mesh = plsc.VectorSubcoreMesh(core_axis_name="c", subcore_axis_name="s")

    @functools.partial(
        pl.kernel,
        out_type=jax.ShapeDtypeStruct((NC, NS, RPT, D), jnp.float32),
        mesh=mesh,
        scratch_types=[
            [pltpu.VMEM((K,), jnp.int32)] * 2,
            [pltpu.VMEM((K,), jnp.int32)] * 2,
            pltpu.VMEM((EPT,), jnp.float32),
            pltpu.VMEM((K, D), jnp.float32),
            pltpu.VMEM((ZR, D), jnp.float32),
            pltpu.VMEM_SHARED((NP, D), jnp.float32),
            pltpu.SemaphoreType.DMA,
            [pltpu.SemaphoreType.DMA] * 2,
        ],
    )
    def k(x_hbm, src_hbm, dst_hbm, ea_hbm, out_hbm,
          src_v, dst_v, ea_all, rows_v, zero_v, acc_sh, gsem, isem):
        cid = lax.axis_index("c")
        sid = lax.axis_index("s")
        wid = cid * NS + sid

        def zrow(i, carry):
            for r in range(D // LANES):
                zero_v[i, pl.ds(r * LANES, LANES)] = jnp.zeros(
                    (LANES,), jnp.float32)
            return carry
        lax.fori_loop(0, ZR, zrow, 0)
        for t in range(RPT // ZR):
            pltpu.sync_copy(zero_v,
                            acc_sh.at[pl.ds(sid * RPT + t * ZR, ZR)])

        ebase = pl.multiple_of(wid * EPT, EPT)
        pltpu.sync_copy(ea_hbm.at[pl.ds(ebase, EPT)], ea_all)
        plsc.subcore_barrier()

        def idx_start(c, b):
            base = pl.multiple_of(wid * EPT + c * K, K)
            pltpu.async_copy(src_hbm.at[pl.ds(base, K)], src_v[b],
                             isem[b])
            pltpu.async_copy(dst_hbm.at[pl.ds(base, K)], dst_v[b],
                             isem[b])

        def idx_wait(b):
            pltpu.make_async_copy(src_hbm.at[pl.ds(0, K)], src_v[b],
                                  isem[b]).wait()
            pltpu.make_async_copy(dst_hbm.at[pl.ds(0, K)], dst_v[b],
                                  isem[b]).wait()

        def run_chunk(c, b, prefetch):
            idx_wait(b)
            if prefetch:
                idx_start(c + 1, 1 - b)
            pltpu.async_copy(x_hbm.at[src_v[b]], rows_v, gsem).wait()

            def scale(g, c3):
                eav = ea_all[pl.ds(c * K + g * LANES, LANES)]
                for li in range(LANES):
                    a = eav[li]
                    j = g * LANES + li
                    for r in range(D // LANES):
                        sl = pl.ds(r * LANES, LANES)
                        rows_v[j, sl] = rows_v[j, sl] * a
                return c3
            lax.fori_loop(0, KG, scale, 0)

            pltpu.sync_copy(rows_v, acc_sh.at[dst_v[b]], add=True)

        idx_start(0, 0)

        def pair(p, carry):
            for b in range(2):
                run_chunk(2 * p + b, b, prefetch=True)
            return carry
        lax.fori_loop(0, CHUNKS // 2, pair, 0)
        run_chunk(CHUNKS - 1, 0, prefetch=False)
        plsc.subcore_barrier()

        pltpu.sync_copy(acc_sh.at[pl.ds(sid * RPT, RPT)],
                        out_hbm.at[cid, sid])

    return k(x, src, dst, ea).reshape(NC, NP, D)


_BN = 400


def _mm_body_relu(p_ref, w_ref, b_ref, o_ref):
    a = p_ref[0] + p_ref[1]
    h = jnp.dot(a, w_ref[...], preferred_element_type=jnp.float32)
    o_ref[...] = jnp.maximum(h + b_ref[...], 0.0)


def _mm_body_lin(p_ref, w_ref, b_ref, o_ref):
    a = p_ref[0] + p_ref[1]
    h = jnp.dot(a, w_ref[...], preferred_element_type=jnp.float32)
    o_ref[...] = h + b_ref[...]


def _mm(p, w, b, relu):
    body = _mm_body_relu if relu else _mm_body_lin
    return pl.pallas_call(
        body,
        grid=(N // _BN,),
        in_specs=[
            pl.BlockSpec((NC, _BN, D), lambda i: (0, i, 0)),
            pl.BlockSpec((D, D), lambda i: (0, 0)),
            pl.BlockSpec((1, D), lambda i: (0, 0)),
        ],
        out_specs=pl.BlockSpec((_BN, D), lambda i: (i, 0)),
        out_shape=jax.ShapeDtypeStruct((N, D), jnp.float32),
    )(p, w, b.reshape(1, D))


def kernel(x, edge_index, edge_attr, W1, b1, W2, b2):
    src = edge_index[0]
    dst = edge_index[1]
    p1 = _sc_agg(x, src, dst, edge_attr)
    h = _mm(p1, W1, b1, relu=True)
    p2 = _sc_agg(h, src, dst, edge_attr)
    return _mm(p2, W2, b2, relu=False)


# speedup vs baseline: 3.2215x; 1.5224x over previous
import functools

import jax
import jax.numpy as jnp
from jax import lax
from jax.experimental import pallas as pl
from jax.experimental.pallas import tpu as pltpu
from jax.experimental.pallas import tpu_sc as plsc

N = 10000
D = 128
E = 320000
LANES = 16

NC = 2
NS = 16
NW = NC * NS
EPT = E // NW
K = 80
CHUNKS = EPT // K
NP = 10240
RPT = NP // NS
ZR = 128
KG = K // LANES


def _sc_agg(x, src, dst, ea):
    mesh = plsc.VectorSubcoreMesh(core_axis_name="c", subcore_axis_name="s")

    @functools.partial(
        pl.kernel,
        out_type=jax.ShapeDtypeStruct((NC, NS, RPT, D), jnp.float32),
        mesh=mesh,
        scratch_types=[
            [pltpu.VMEM((K,), jnp.int32)] * 4,
            [pltpu.VMEM((K,), jnp.int32)] * 4,
            pltpu.VMEM((EPT,), jnp.float32),
            [pltpu.VMEM((K, D), jnp.float32)] * 2,
            pltpu.VMEM_SHARED((NP, D), jnp.float32),
            [pltpu.SemaphoreType.DMA] * 2,
            [pltpu.SemaphoreType.DMA] * 4,
        ],
    )
    def k(x_hbm, src_hbm, dst_hbm, ea_hbm, out_hbm,
          src_v, dst_v, ea_all, rows, acc_sh, gsem, isem):
        cid = lax.axis_index("c")
        sid = lax.axis_index("s")
        wid = cid * NS + sid

        def zrow(i, carry):
            for r in range(D // LANES):
                sl = pl.ds(r * LANES, LANES)
                rows[0][i, sl] = jnp.zeros((LANES,), jnp.float32)
                rows[1][i, sl] = jnp.zeros((LANES,), jnp.float32)
            return carry
        lax.fori_loop(0, K, zrow, 0)
        for t in range(RPT // K):
            pltpu.sync_copy(rows[t % 2],
                            acc_sh.at[pl.ds(sid * RPT + t * K, K)])

        ebase = pl.multiple_of(wid * EPT, EPT)
        pltpu.sync_copy(ea_hbm.at[pl.ds(ebase, EPT)], ea_all)
        plsc.subcore_barrier()

        def idx_start(c, b):
            base = pl.multiple_of(wid * EPT + c * K, K)
            pltpu.async_copy(src_hbm.at[pl.ds(base, K)], src_v[b],
                             isem[b])
            pltpu.async_copy(dst_hbm.at[pl.ds(base, K)], dst_v[b],
                             isem[b])

        def idx_wait(b):
            pltpu.make_async_copy(src_hbm.at[pl.ds(0, K)], src_v[b],
                                  isem[b]).wait()
            pltpu.make_async_copy(dst_hbm.at[pl.ds(0, K)], dst_v[b],
                                  isem[b]).wait()

        def gather_start(b, r):
            pltpu.async_copy(x_hbm.at[src_v[b]], rows[r], gsem[r])

        def gather_wait(b, r):
            pltpu.make_async_copy(x_hbm.at[src_v[b]], rows[r],
                                  gsem[r]).wait()

        def run_chunk(c, b, r, steady):
            gather_wait(b, r)
            if steady:
                idx_wait((b + 1) % 4)
                gather_start((b + 1) % 4, 1 - r)

                @pl.when(c + 3 < CHUNKS)
                def _():
                    idx_start(c + 3, (b + 3) % 4)

            def scale(g, c3):
                eav = ea_all[pl.ds(c * K + g * LANES, LANES)]
                for li in range(LANES):
                    a = eav[li]
                    j = g * LANES + li
                    for q in range(D // LANES):
                        sl = pl.ds(q * LANES, LANES)
                        rows[r][j, sl] = rows[r][j, sl] * a
                return c3
            lax.fori_loop(0, KG, scale, 0)

            pltpu.sync_copy(rows[r], acc_sh.at[dst_v[b]], add=True)

        idx_start(0, 0)
        idx_start(1, 1)
        idx_start(2, 2)
        idx_wait(0)
        gather_start(0, 0)

        def quad(p, carry):
            for b in range(4):
                run_chunk(4 * p + b, b, b % 2, steady=True)
            return carry
        lax.fori_loop(0, (CHUNKS - 1) // 4, quad, 0)
        run_chunk(CHUNKS - 1, (CHUNKS - 1) % 4, (CHUNKS - 1) % 2,
                  steady=False)
        plsc.subcore_barrier()

        pltpu.sync_copy(acc_sh.at[pl.ds(sid * RPT, RPT)],
                        out_hbm.at[cid, sid])

    return k(x, src, dst, ea).reshape(NC, NP, D)


_BN = 400


def _mm_body_relu(p_ref, w_ref, b_ref, o_ref):
    a = p_ref[0] + p_ref[1]
    h = jnp.dot(a, w_ref[...], preferred_element_type=jnp.float32)
    o_ref[...] = jnp.maximum(h + b_ref[...], 0.0)


def _mm_body_lin(p_ref, w_ref, b_ref, o_ref):
    a = p_ref[0] + p_ref[1]
    h = jnp.dot(a, w_ref[...], preferred_element_type=jnp.float32)
    o_ref[...] = h + b_ref[...]


def _mm(p, w, b, relu):
    body = _mm_body_relu if relu else _mm_body_lin
    return pl.pallas_call(
        body,
        grid=(N // _BN,),
        in_specs=[
            pl.BlockSpec((NC, _BN, D), lambda i: (0, i, 0)),
            pl.BlockSpec((D, D), lambda i: (0, 0)),
            pl.BlockSpec((1, D), lambda i: (0, 0)),
        ],
        out_specs=pl.BlockSpec((_BN, D), lambda i: (i, 0)),
        out_shape=jax.ShapeDtypeStruct((N, D), jnp.float32),
    )(p, w, b.reshape(1, D))


def kernel(x, edge_index, edge_attr, W1, b1, W2, b2):
    src = edge_index[0]
    dst = edge_index[1]
    p1 = _sc_agg(x, src, dst, edge_attr)
    h = _mm(p1, W1, b1, relu=True)
    p2 = _sc_agg(h, src, dst, edge_attr)
    return _mm(p2, W2, b2, relu=False)


# speedup vs baseline: 3.2287x; 1.0022x over previous
import functools

import jax
import jax.numpy as jnp
from jax import lax
from jax.experimental import pallas as pl
from jax.experimental.pallas import tpu as pltpu
from jax.experimental.pallas import tpu_sc as plsc

N = 10000
D = 128
E = 320000
LANES = 16

NC = 2
NS = 16
NW = NC * NS
EPT = E // NW
K = 80
CHUNKS = EPT // K
NP = 10240
RPT = NP // NS
ZR = 128
KG = K // LANES


def _sc_agg(x, src, dst, ea):
    mesh = plsc.VectorSubcoreMesh(core_axis_name="c", subcore_axis_name="s")

    @functools.partial(
        pl.kernel,
        out_type=jax.ShapeDtypeStruct((NC, NS, RPT, D), jnp.float32),
        mesh=mesh,
        scratch_types=[
            [pltpu.VMEM((K,), jnp.int32)] * 4,
            [pltpu.VMEM((K,), jnp.int32)] * 4,
            [pltpu.VMEM((K,), jnp.float32)] * 4,
            [pltpu.VMEM((K, D), jnp.float32)] * 4,
            pltpu.VMEM_SHARED((NP, D), jnp.float32),
            [pltpu.SemaphoreType.DMA] * 4,
            [pltpu.SemaphoreType.DMA] * 4,
            [pltpu.SemaphoreType.DMA] * 4,
        ],
    )
    def k(x_hbm, src_hbm, dst_hbm, ea_hbm, out_hbm,
          src_v, dst_v, ea_v, rows, acc_sh, gsem, isem, ssem):
        cid = lax.axis_index("c")
        sid = lax.axis_index("s")
        wid = cid * NS + sid

        def zrow(i, carry):
            for r in range(D // LANES):
                sl = pl.ds(r * LANES, LANES)
                rows[0][i, sl] = jnp.zeros((LANES,), jnp.float32)
                rows[1][i, sl] = jnp.zeros((LANES,), jnp.float32)
            return carry
        lax.fori_loop(0, K, zrow, 0)
        for t in range(RPT // K):
            pltpu.sync_copy(rows[t % 2],
                            acc_sh.at[pl.ds(sid * RPT + t * K, K)])
        plsc.subcore_barrier()

        def idx_start(c, b):
            base = pl.multiple_of(wid * EPT + c * K, K)
            pltpu.async_copy(src_hbm.at[pl.ds(base, K)], src_v[b],
                             isem[b])
            pltpu.async_copy(dst_hbm.at[pl.ds(base, K)], dst_v[b],
                             isem[b])
            pltpu.async_copy(ea_hbm.at[pl.ds(base, K)], ea_v[b],
                             isem[b])

        def idx_wait(b):
            pltpu.make_async_copy(src_hbm.at[pl.ds(0, K)], src_v[b],
                                  isem[b]).wait()
            pltpu.make_async_copy(dst_hbm.at[pl.ds(0, K)], dst_v[b],
                                  isem[b]).wait()
            pltpu.make_async_copy(ea_hbm.at[pl.ds(0, K)], ea_v[b],
                                  isem[b]).wait()

        def gather_start(b):
            pltpu.async_copy(x_hbm.at[src_v[b]], rows[b], gsem[b])

        def gather_wait(b):
            pltpu.make_async_copy(x_hbm.at[src_v[b]], rows[b],
                                  gsem[b]).wait()

        def scatter_start(b):
            pltpu.async_copy(rows[b], acc_sh.at[dst_v[b]], ssem[b],
                             add=True)

        def scatter_wait(b):
            pltpu.make_async_copy(rows[b], acc_sh.at[dst_v[b]],
                                  ssem[b]).wait()

        def run_chunk(c, b, steady):
            gather_wait(b)
            if steady:
                idx_wait((b + 1) % 4)
                @pl.when(c >= 2)
                def _():
                    scatter_wait((b + 2) % 4)
                gather_start((b + 1) % 4)

                @pl.when(c + 2 < CHUNKS)
                def _():
                    idx_start(c + 2, (b + 2) % 4)

            def scale(g, c3):
                eav = ea_v[b][pl.ds(g * LANES, LANES)]
                for li in range(LANES):
                    a = eav[li]
                    j = g * LANES + li
                    for q in range(D // LANES):
                        sl = pl.ds(q * LANES, LANES)
                        rows[b][j, sl] = rows[b][j, sl] * a
                return c3
            lax.fori_loop(0, KG, scale, 0)

            scatter_start(b)

        idx_start(0, 0)
        idx_start(1, 1)
        idx_wait(0)
        gather_start(0)

        def quad(p, carry):
            for b in range(4):
                run_chunk(4 * p + b, b, steady=True)
            return carry
        lax.fori_loop(0, (CHUNKS - 1) // 4, quad, 0)
        run_chunk(CHUNKS - 1, (CHUNKS - 1) % 4, steady=False)
        scatter_wait((CHUNKS - 3) % 4)
        scatter_wait((CHUNKS - 2) % 4)
        scatter_wait((CHUNKS - 1) % 4)
        plsc.subcore_barrier()

        pltpu.sync_copy(acc_sh.at[pl.ds(sid * RPT, RPT)],
                        out_hbm.at[cid, sid])

    return k(x, src, dst, ea).reshape(NC, NP, D)


_BN = 400


def _mm_body_relu(p_ref, w_ref, b_ref, o_ref):
    a = p_ref[0] + p_ref[1]
    h = jnp.dot(a, w_ref[...], preferred_element_type=jnp.float32)
    o_ref[...] = jnp.maximum(h + b_ref[...], 0.0)


def _mm_body_lin(p_ref, w_ref, b_ref, o_ref):
    a = p_ref[0] + p_ref[1]
    h = jnp.dot(a, w_ref[...], preferred_element_type=jnp.float32)
    o_ref[...] = h + b_ref[...]


def _mm(p, w, b, relu):
    body = _mm_body_relu if relu else _mm_body_lin
    return pl.pallas_call(
        body,
        grid=(N // _BN,),
        in_specs=[
            pl.BlockSpec((NC, _BN, D), lambda i: (0, i, 0)),
            pl.BlockSpec((D, D), lambda i: (0, 0)),
            pl.BlockSpec((1, D), lambda i: (0, 0)),
        ],
        out_specs=pl.BlockSpec((_BN, D), lambda i: (i, 0)),
        out_shape=jax.ShapeDtypeStruct((N, D), jnp.float32),
    )(p, w, b.reshape(1, D))


def kernel(x, edge_index, edge_attr, W1, b1, W2, b2):
    src = edge_index[0]
    dst = edge_index[1]
    p1 = _sc_agg(x, src, dst, edge_attr)
    h = _mm(p1, W1, b1, relu=True)
    p2 = _sc_agg(h, src, dst, edge_attr)
    return _mm(p2, W2, b2, relu=False)
